# baseline (speedup 1.0000x reference)
import numpy as np
import jax
import jax.numpy as jnp
from jax.experimental import pallas as pl

L = 2
B = 8
H = 224
W_IMG = 224
PATCH = 14
PH = 16
PW = 16
P = PH * PW
D = 1024
C_TOK = 3 * PATCH * PATCH


def _build_a3() -> np.ndarray:
    a = np.zeros((P, P), np.float32)
    for i in range(PH):
        for j in range(PW):
            p = i * PW + j
            nbrs = [(i + di, j + dj)
                    for di in (-1, 0, 1) for dj in (-1, 0, 1)
                    if 0 <= i + di < PH and 0 <= j + dj < PW]
            w = 1.0 / len(nbrs)
            for (y, x) in nbrs:
                a[p, y * PW + x] += w
    return a


def _build_resize_mat(n_in: int, n_out: int) -> np.ndarray:
    scale = n_in / n_out
    r = np.zeros((n_out, n_in), np.float32)
    for y in range(n_out):
        s = (y + 0.5) * scale - 0.5
        w = np.maximum(0.0, 1.0 - np.abs(s - np.arange(n_in)))
        r[y] = w / w.sum()
    return r.astype(np.float32)


_A3 = _build_a3()
_RMAT = _build_resize_mat(PH, H)


def _feat_kernel(t_ref, w_ref, a3_ref, out_ref):
    tok = t_ref[0]
    wl = w_ref[0]
    f = jnp.dot(tok, wl, preferred_element_type=jnp.float32)
    out_ref[0, 0, 0] = f
    out_ref[1, 0, 0] = jnp.dot(a3_ref[...], f,
                               preferred_element_type=jnp.float32)


def _msm_kernel(x_ref, acc_ref):
    lr = pl.program_id(0)
    b = pl.program_id(1)
    rows = x_ref[0, pl.ds(b * P, P), :]
    x2r = jnp.sum(rows * rows, axis=1, keepdims=True)
    m1 = jnp.full((P, 1), 1e9, jnp.float32)
    m2 = m1
    for c in range(B):
        cols = x_ref[0, c * P:(c + 1) * P, :]
        g = jax.lax.dot_general(rows, cols, (((1,), (1,)), ((), ())),
                                preferred_element_type=jnp.float32)
        x2c = jnp.sum(cols * cols, axis=1)
        t = x2c[None, :] - 2.0 * g
        mc = jnp.min(t, axis=1, keepdims=True) + x2r
        dc = jnp.sqrt(jnp.maximum(mc, 1e-12))
        dc = jnp.where(c == b, 1e9, dc)
        nm1 = jnp.minimum(m1, dc)
        m2 = jnp.minimum(m2, jnp.maximum(m1, dc))
        m1 = nm1
    val = (m1 + m2) * 0.125
    onehot = (jax.lax.broadcasted_iota(jnp.int32, (1, B), 1) == b
              ).astype(jnp.float32)
    contrib = val * onehot

    @pl.when(jnp.logical_and(lr == 0, b == 0))
    def _():
        acc_ref[...] = contrib

    @pl.when(jnp.logical_or(lr != 0, b != 0))
    def _():
        acc_ref[...] = acc_ref[...] + contrib


def _resize_kernel(s_ref, r_ref, maps_ref, score_ref):
    s = s_ref[0]
    rm = r_ref[...]
    tmp = jnp.dot(rm, s, preferred_element_type=jnp.float32)
    m = jax.lax.dot_general(tmp, rm, (((1,), (1,)), ((), ())),
                            preferred_element_type=jnp.float32)
    maps_ref[0] = m
    b = pl.program_id(0)
    onehot = (jax.lax.broadcasted_iota(jnp.int32, (B, 1), 0) == b
              ).astype(jnp.float32)
    contrib = jnp.max(m) * onehot

    @pl.when(b == 0)
    def _():
        score_ref[...] = contrib

    @pl.when(b != 0)
    def _():
        score_ref[...] = score_ref[...] + contrib


def kernel(pixel_values, W):
    tokens = pixel_values.reshape(B, 3, PH, PATCH, PW, PATCH)
    tokens = tokens.transpose(0, 2, 4, 1, 3, 5).reshape(B, P, C_TOK)

    a3 = jnp.asarray(_A3)
    rmat = jnp.asarray(_RMAT)

    x = pl.pallas_call(
        _feat_kernel,
        grid=(L, B),
        in_specs=[
            pl.BlockSpec((1, P, C_TOK), lambda l, b: (b, 0, 0)),
            pl.BlockSpec((1, C_TOK, D), lambda l, b: (l, 0, 0)),
            pl.BlockSpec((P, P), lambda l, b: (0, 0)),
        ],
        out_specs=pl.BlockSpec((2, 1, 1, P, D), lambda l, b: (0, l, b, 0, 0)),
        out_shape=jax.ShapeDtypeStruct((2, L, B, P, D), jnp.float32),
    )(tokens, W, a3)
    x = x.reshape(2 * L, B * P, D)

    acc = pl.pallas_call(
        _msm_kernel,
        grid=(2 * L, B),
        in_specs=[pl.BlockSpec((1, B * P, D), lambda lr, b: (lr, 0, 0))],
        out_specs=pl.BlockSpec((P, B), lambda lr, b: (0, 0)),
        out_shape=jax.ShapeDtypeStruct((P, B), jnp.float32),
    )(x)
    patch_scores = acc.T.reshape(B, PH, PW)

    maps, scores = pl.pallas_call(
        _resize_kernel,
        grid=(B,),
        in_specs=[
            pl.BlockSpec((1, PH, PW), lambda b: (b, 0, 0)),
            pl.BlockSpec((H, PH), lambda b: (0, 0)),
        ],
        out_specs=[
            pl.BlockSpec((1, H, W_IMG), lambda b: (b, 0, 0)),
            pl.BlockSpec((B, 1), lambda b: (0, 0)),
        ],
        out_shape=[
            jax.ShapeDtypeStruct((B, H, W_IMG), jnp.float32),
            jax.ShapeDtypeStruct((B, 1), jnp.float32),
        ],
    )(patch_scores, rmat)
    return scores.reshape(B), maps


# speedup vs baseline: 21.3435x; 21.3435x over previous
import numpy as np
import jax
import jax.numpy as jnp
from jax.experimental import pallas as pl

L = 2
B = 8
H = 224
W_IMG = 224
PATCH = 14
PH = 16
PW = 16
P = PH * PW
D = 1024
C_TOK = 3 * PATCH * PATCH


def _build_a3() -> np.ndarray:
    a = np.zeros((P, P), np.float32)
    for i in range(PH):
        for j in range(PW):
            p = i * PW + j
            nbrs = [(i + di, j + dj)
                    for di in (-1, 0, 1) for dj in (-1, 0, 1)
                    if 0 <= i + di < PH and 0 <= j + dj < PW]
            w = 1.0 / len(nbrs)
            for (y, x) in nbrs:
                a[p, y * PW + x] += w
    return a


def _build_resize_mat(n_in: int, n_out: int) -> np.ndarray:
    scale = n_in / n_out
    r = np.zeros((n_out, n_in), np.float32)
    for y in range(n_out):
        s = (y + 0.5) * scale - 0.5
        w = np.maximum(0.0, 1.0 - np.abs(s - np.arange(n_in)))
        r[y] = w / w.sum()
    return r.astype(np.float32)


_A3 = _build_a3()
_RMAT = _build_resize_mat(PH, H)


def _feat_kernel(t_ref, w_ref, a3_ref, x_ref, xt_ref):
    tok = t_ref[0]
    wl = w_ref[0]
    f = jnp.dot(tok, wl, preferred_element_type=jnp.float32)
    f3 = jnp.dot(a3_ref[...], f, preferred_element_type=jnp.float32)
    x_ref[0, 0, 0] = f
    x_ref[1, 0, 0] = f3
    xt_ref[0, 0] = f.T
    xt_ref[1, 0] = f3.T


def _msm_kernel(x_ref, xt_ref, acc_ref):
    lr = pl.program_id(0)
    b = pl.program_id(1)
    rows = x_ref[0, pl.ds(b * P, P), :]
    xt = xt_ref[0]
    x2r = jnp.sum(rows * rows, axis=1, keepdims=True)
    x2all = jnp.sum(xt * xt, axis=0, keepdims=True)
    g = jnp.dot(rows, xt, preferred_element_type=jnp.float32)
    t = x2all - 2.0 * g
    m1 = jnp.full((P, 1), 1e9, jnp.float32)
    m2 = m1
    for c in range(B):
        mc = jnp.min(t[:, c * P:(c + 1) * P], axis=1, keepdims=True) + x2r
        dc = jnp.sqrt(jnp.maximum(mc, 1e-12))
        dc = jnp.where(c == b, 1e9, dc)
        nm1 = jnp.minimum(m1, dc)
        m2 = jnp.minimum(m2, jnp.maximum(m1, dc))
        m1 = nm1
    val = (m1 + m2) * 0.125
    onehot = (jax.lax.broadcasted_iota(jnp.int32, (1, B), 1) == b
              ).astype(jnp.float32)
    contrib = val * onehot

    @pl.when(jnp.logical_and(lr == 0, b == 0))
    def _():
        acc_ref[...] = contrib

    @pl.when(jnp.logical_or(lr != 0, b != 0))
    def _():
        acc_ref[...] = acc_ref[...] + contrib


def _resize_kernel(s_ref, r_ref, maps_ref, score_ref):
    s = s_ref[0]
    rm = r_ref[...]
    tmp = jnp.dot(rm, s, preferred_element_type=jnp.float32)
    m = jax.lax.dot_general(tmp, rm, (((1,), (1,)), ((), ())),
                            preferred_element_type=jnp.float32)
    maps_ref[0] = m
    b = pl.program_id(0)
    onehot = (jax.lax.broadcasted_iota(jnp.int32, (B, 1), 0) == b
              ).astype(jnp.float32)
    contrib = jnp.max(m) * onehot

    @pl.when(b == 0)
    def _():
        score_ref[...] = contrib

    @pl.when(b != 0)
    def _():
        score_ref[...] = score_ref[...] + contrib


def kernel(pixel_values, W):
    tokens = pixel_values.reshape(B, 3, PH, PATCH, PW, PATCH)
    tokens = tokens.transpose(0, 2, 4, 1, 3, 5).reshape(B, P, C_TOK)

    a3 = jnp.asarray(_A3)
    rmat = jnp.asarray(_RMAT)

    x, xt = pl.pallas_call(
        _feat_kernel,
        grid=(L, B),
        in_specs=[
            pl.BlockSpec((1, P, C_TOK), lambda l, b: (b, 0, 0)),
            pl.BlockSpec((1, C_TOK, D), lambda l, b: (l, 0, 0)),
            pl.BlockSpec((P, P), lambda l, b: (0, 0)),
        ],
        out_specs=[
            pl.BlockSpec((2, 1, 1, P, D), lambda l, b: (0, l, b, 0, 0)),
            pl.BlockSpec((2, 1, D, P), lambda l, b: (0, l, 0, b)),
        ],
        out_shape=[
            jax.ShapeDtypeStruct((2, L, B, P, D), jnp.float32),
            jax.ShapeDtypeStruct((2, L, D, B * P), jnp.float32),
        ],
    )(tokens, W, a3)
    x = x.reshape(2 * L, B * P, D)
    xt = xt.reshape(2 * L, D, B * P)

    acc = pl.pallas_call(
        _msm_kernel,
        grid=(2 * L, B),
        in_specs=[
            pl.BlockSpec((1, B * P, D), lambda lr, b: (lr, 0, 0)),
            pl.BlockSpec((1, D, B * P), lambda lr, b: (lr, 0, 0)),
        ],
        out_specs=pl.BlockSpec((P, B), lambda lr, b: (0, 0)),
        out_shape=jax.ShapeDtypeStruct((P, B), jnp.float32),
    )(x, xt)
    patch_scores = acc.T.reshape(B, PH, PW)

    maps, scores = pl.pallas_call(
        _resize_kernel,
        grid=(B,),
        in_specs=[
            pl.BlockSpec((1, PH, PW), lambda b: (b, 0, 0)),
            pl.BlockSpec((H, PH), lambda b: (0, 0)),
        ],
        out_specs=[
            pl.BlockSpec((1, H, W_IMG), lambda b: (b, 0, 0)),
            pl.BlockSpec((B, 1), lambda b: (0, 0)),
        ],
        out_shape=[
            jax.ShapeDtypeStruct((B, H, W_IMG), jnp.float32),
            jax.ShapeDtypeStruct((B, 1), jnp.float32),
        ],
    )(patch_scores, rmat)
    return scores.reshape(B), maps


# speedup vs baseline: 26.8714x; 1.2590x over previous
import numpy as np
import jax
import jax.numpy as jnp
from jax.experimental import pallas as pl

L = 2
B = 8
H = 224
W_IMG = 224
PATCH = 14
PH = 16
PW = 16
P = PH * PW
D = 1024
C_TOK = 3 * PATCH * PATCH


def _build_a3() -> np.ndarray:
    a = np.zeros((P, P), np.float32)
    for i in range(PH):
        for j in range(PW):
            p = i * PW + j
            nbrs = [(i + di, j + dj)
                    for di in (-1, 0, 1) for dj in (-1, 0, 1)
                    if 0 <= i + di < PH and 0 <= j + dj < PW]
            w = 1.0 / len(nbrs)
            for (y, x) in nbrs:
                a[p, y * PW + x] += w
    return a


def _build_resize_mat(n_in: int, n_out: int) -> np.ndarray:
    scale = n_in / n_out
    r = np.zeros((n_out, n_in), np.float32)
    for y in range(n_out):
        s = (y + 0.5) * scale - 0.5
        w = np.maximum(0.0, 1.0 - np.abs(s - np.arange(n_in)))
        r[y] = w / w.sum()
    return r.astype(np.float32)


_A3 = _build_a3()
_RMAT = _build_resize_mat(PH, H)


def _feat_kernel(t_ref, w_ref, a3_ref, x_ref, xt_ref):
    tok = t_ref[0]
    wl = w_ref[0]
    f = jnp.dot(tok, wl, preferred_element_type=jnp.float32)
    f3 = jnp.dot(a3_ref[...], f, preferred_element_type=jnp.float32)
    x_ref[0, 0, 0] = f
    x_ref[1, 0, 0] = f3
    xt_ref[0, 0] = f.T
    xt_ref[1, 0] = f3.T


def _two_min_update(m1, m2, v):
    nm1 = jnp.minimum(m1, v)
    nm2 = jnp.minimum(m2, jnp.maximum(m1, v))
    return nm1, nm2


def _msm_kernel(x_ref, xt_ref, acc_ref):
    lr = pl.program_id(0)
    big = jnp.float32(1e9)

    x2c = [jnp.sum(x_ref[0, b * P:(b + 1) * P, :] ** 2, axis=1, keepdims=True)
           for b in range(B)]
    x2r = [jnp.sum(xt_ref[0, :, b * P:(b + 1) * P] ** 2, axis=0, keepdims=True)
           for b in range(B)]

    m1r = [jnp.full((P, 1), big) for _ in range(B)]
    m2r = [jnp.full((P, 1), big) for _ in range(B)]
    m1c = [jnp.full((1, P), big) for _ in range(B)]
    m2c = [jnp.full((1, P), big) for _ in range(B)]

    for b in range(B - 1):
        rows = x_ref[0, b * P:(b + 1) * P, :]
        xt_s = xt_ref[0, :, (b + 1) * P:]
        g = jnp.dot(rows, xt_s, preferred_element_type=jnp.float32)
        for j, c in enumerate(range(b + 1, B)):
            gc = g[:, j * P:(j + 1) * P]
            mb = jnp.min(x2r[c] - 2.0 * gc, axis=1, keepdims=True) + x2c[b]
            m1r[b], m2r[b] = _two_min_update(m1r[b], m2r[b], mb)
            mc = jnp.min(x2c[b] - 2.0 * gc, axis=0, keepdims=True) + x2r[c]
            m1c[c], m2c[c] = _two_min_update(m1c[c], m2c[c], mc)

    cols = []
    for b in range(B):
        m1ct = m1c[b].T
        m2ct = m2c[b].T
        m1 = jnp.minimum(m1r[b], m1ct)
        m2 = jnp.minimum(jnp.maximum(m1r[b], m1ct), jnp.minimum(m2r[b], m2ct))
        d1 = jnp.sqrt(jnp.maximum(m1, 1e-12))
        d2 = jnp.sqrt(jnp.maximum(m2, 1e-12))
        cols.append((d1 + d2) * 0.125)
    contrib = jnp.concatenate(cols, axis=1)

    @pl.when(lr == 0)
    def _():
        acc_ref[...] = contrib

    @pl.when(lr != 0)
    def _():
        acc_ref[...] = acc_ref[...] + contrib


def _resize_kernel(s_ref, r_ref, maps_ref, score_ref):
    s = s_ref[0]
    rm = r_ref[...]
    tmp = jnp.dot(rm, s, preferred_element_type=jnp.float32)
    m = jax.lax.dot_general(tmp, rm, (((1,), (1,)), ((), ())),
                            preferred_element_type=jnp.float32)
    maps_ref[0] = m
    b = pl.program_id(0)
    onehot = (jax.lax.broadcasted_iota(jnp.int32, (B, 1), 0) == b
              ).astype(jnp.float32)
    contrib = jnp.max(m) * onehot

    @pl.when(b == 0)
    def _():
        score_ref[...] = contrib

    @pl.when(b != 0)
    def _():
        score_ref[...] = score_ref[...] + contrib


def kernel(pixel_values, W):
    tokens = pixel_values.reshape(B, 3, PH, PATCH, PW, PATCH)
    tokens = tokens.transpose(0, 2, 4, 1, 3, 5).reshape(B, P, C_TOK)

    a3 = jnp.asarray(_A3)
    rmat = jnp.asarray(_RMAT)

    x, xt = pl.pallas_call(
        _feat_kernel,
        grid=(L, B),
        in_specs=[
            pl.BlockSpec((1, P, C_TOK), lambda l, b: (b, 0, 0)),
            pl.BlockSpec((1, C_TOK, D), lambda l, b: (l, 0, 0)),
            pl.BlockSpec((P, P), lambda l, b: (0, 0)),
        ],
        out_specs=[
            pl.BlockSpec((2, 1, 1, P, D), lambda l, b: (0, l, b, 0, 0)),
            pl.BlockSpec((2, 1, D, P), lambda l, b: (0, l, 0, b)),
        ],
        out_shape=[
            jax.ShapeDtypeStruct((2, L, B, P, D), jnp.float32),
            jax.ShapeDtypeStruct((2, L, D, B * P), jnp.float32),
        ],
    )(tokens, W, a3)
    x = x.reshape(2 * L, B * P, D)
    xt = xt.reshape(2 * L, D, B * P)

    acc = pl.pallas_call(
        _msm_kernel,
        grid=(2 * L,),
        in_specs=[
            pl.BlockSpec((1, B * P, D), lambda lr: (lr, 0, 0)),
            pl.BlockSpec((1, D, B * P), lambda lr: (lr, 0, 0)),
        ],
        out_specs=pl.BlockSpec((P, B), lambda lr: (0, 0)),
        out_shape=jax.ShapeDtypeStruct((P, B), jnp.float32),
    )(x, xt)
    patch_scores = acc.T.reshape(B, PH, PW)

    maps, scores = pl.pallas_call(
        _resize_kernel,
        grid=(B,),
        in_specs=[
            pl.BlockSpec((1, PH, PW), lambda b: (b, 0, 0)),
            pl.BlockSpec((H, PH), lambda b: (0, 0)),
        ],
        out_specs=[
            pl.BlockSpec((1, H, W_IMG), lambda b: (b, 0, 0)),
            pl.BlockSpec((B, 1), lambda b: (0, 0)),
        ],
        out_shape=[
            jax.ShapeDtypeStruct((B, H, W_IMG), jnp.float32),
            jax.ShapeDtypeStruct((B, 1), jnp.float32),
        ],
    )(patch_scores, rmat)
    return scores.reshape(B), maps


# speedup vs baseline: 27.8339x; 1.0358x over previous
import numpy as np
import jax
import jax.numpy as jnp
from jax.experimental import pallas as pl
from jax.experimental.pallas import tpu as pltpu

L = 2
B = 8
H = 224
W_IMG = 224
PATCH = 14
PH = 16
PW = 16
P = PH * PW
D = 1024
C_TOK = 3 * PATCH * PATCH


def _build_a3() -> np.ndarray:
    a = np.zeros((P, P), np.float32)
    for i in range(PH):
        for j in range(PW):
            p = i * PW + j
            nbrs = [(i + di, j + dj)
                    for di in (-1, 0, 1) for dj in (-1, 0, 1)
                    if 0 <= i + di < PH and 0 <= j + dj < PW]
            w = 1.0 / len(nbrs)
            for (y, x) in nbrs:
                a[p, y * PW + x] += w
    return a


def _build_resize_mat(n_in: int, n_out: int) -> np.ndarray:
    scale = n_in / n_out
    r = np.zeros((n_out, n_in), np.float32)
    for y in range(n_out):
        s = (y + 0.5) * scale - 0.5
        w = np.maximum(0.0, 1.0 - np.abs(s - np.arange(n_in)))
        r[y] = w / w.sum()
    return r.astype(np.float32)


_A3 = _build_a3()
_RMAT = _build_resize_mat(PH, H)


def _two_min_update(m1, m2, v):
    nm1 = jnp.minimum(m1, v)
    nm2 = jnp.minimum(m2, jnp.maximum(m1, v))
    return nm1, nm2


def _pair_msm(x_ref, xt_ref):
    big = jnp.float32(1e9)
    x2c = [jnp.sum(x_ref[b * P:(b + 1) * P, :] ** 2, axis=1, keepdims=True)
           for b in range(B)]
    x2r = [jnp.sum(xt_ref[:, b * P:(b + 1) * P] ** 2, axis=0, keepdims=True)
           for b in range(B)]

    m1r = [jnp.full((P, 1), big) for _ in range(B)]
    m2r = [jnp.full((P, 1), big) for _ in range(B)]
    m1c = [jnp.full((1, P), big) for _ in range(B)]
    m2c = [jnp.full((1, P), big) for _ in range(B)]

    for b in range(B - 1):
        rows = x_ref[b * P:(b + 1) * P, :]
        xt_s = xt_ref[:, (b + 1) * P:]
        g = jnp.dot(rows, xt_s, preferred_element_type=jnp.float32)
        for j, c in enumerate(range(b + 1, B)):
            gc = g[:, j * P:(j + 1) * P]
            mb = jnp.min(x2r[c] - 2.0 * gc, axis=1, keepdims=True) + x2c[b]
            m1r[b], m2r[b] = _two_min_update(m1r[b], m2r[b], mb)
            mc = jnp.min(x2c[b] - 2.0 * gc, axis=0, keepdims=True) + x2r[c]
            m1c[c], m2c[c] = _two_min_update(m1c[c], m2c[c], mc)

    cols = []
    for b in range(B):
        m1ct = m1c[b].T
        m2ct = m2c[b].T
        m1 = jnp.minimum(m1r[b], m1ct)
        m2 = jnp.minimum(jnp.maximum(m1r[b], m1ct), jnp.minimum(m2r[b], m2ct))
        d1 = jnp.sqrt(jnp.maximum(m1, 1e-12))
        d2 = jnp.sqrt(jnp.maximum(m2, 1e-12))
        cols.append((d1 + d2) * 0.125)
    return jnp.concatenate(cols, axis=1)


def _fused_kernel(t2d_ref, tt_ref, w_ref, wt_ref, a3_ref, a3t_ref,
                  acc_ref, f_ref, ft_ref, f3_ref, f3t_ref):
    l = pl.program_id(0)
    f_ref[...] = jnp.dot(t2d_ref[...], w_ref[0],
                         preferred_element_type=jnp.float32)
    ft_ref[...] = jnp.dot(wt_ref[0], tt_ref[...],
                          preferred_element_type=jnp.float32)
    for b in range(B):
        f3_ref[b * P:(b + 1) * P, :] = jnp.dot(
            a3_ref[...], f_ref[b * P:(b + 1) * P, :],
            preferred_element_type=jnp.float32)
        f3t_ref[:, b * P:(b + 1) * P] = jnp.dot(
            ft_ref[:, b * P:(b + 1) * P], a3t_ref[...],
            preferred_element_type=jnp.float32)
    contrib = _pair_msm(f_ref, ft_ref) + _pair_msm(f3_ref, f3t_ref)

    @pl.when(l == 0)
    def _():
        acc_ref[...] = contrib

    @pl.when(l != 0)
    def _():
        acc_ref[...] = acc_ref[...] + contrib


def _resize_kernel(s_ref, r_ref, maps_ref, score_ref):
    s = s_ref[0]
    rm = r_ref[...]
    tmp = jnp.dot(rm, s, preferred_element_type=jnp.float32)
    m = jax.lax.dot_general(tmp, rm, (((1,), (1,)), ((), ())),
                            preferred_element_type=jnp.float32)
    maps_ref[0] = m
    b = pl.program_id(0)
    onehot = (jax.lax.broadcasted_iota(jnp.int32, (B, 1), 0) == b
              ).astype(jnp.float32)
    contrib = jnp.max(m) * onehot

    @pl.when(b == 0)
    def _():
        score_ref[...] = contrib

    @pl.when(b != 0)
    def _():
        score_ref[...] = score_ref[...] + contrib


def kernel(pixel_values, W):
    tokens = pixel_values.reshape(B, 3, PH, PATCH, PW, PATCH)
    tokens = tokens.transpose(0, 2, 4, 1, 3, 5).reshape(B * P, C_TOK)
    tokens_t = tokens.T
    w_t = jnp.swapaxes(W, 1, 2)

    a3 = jnp.asarray(_A3)
    a3t = a3.T
    rmat = jnp.asarray(_RMAT)

    acc = pl.pallas_call(
        _fused_kernel,
        grid=(L,),
        in_specs=[
            pl.BlockSpec((B * P, C_TOK), lambda l: (0, 0)),
            pl.BlockSpec((C_TOK, B * P), lambda l: (0, 0)),
            pl.BlockSpec((1, C_TOK, D), lambda l: (l, 0, 0)),
            pl.BlockSpec((1, D, C_TOK), lambda l: (l, 0, 0)),
            pl.BlockSpec((P, P), lambda l: (0, 0)),
            pl.BlockSpec((P, P), lambda l: (0, 0)),
        ],
        out_specs=pl.BlockSpec((P, B), lambda l: (0, 0)),
        out_shape=jax.ShapeDtypeStruct((P, B), jnp.float32),
        scratch_shapes=[
            pltpu.VMEM((B * P, D), jnp.float32),
            pltpu.VMEM((D, B * P), jnp.float32),
            pltpu.VMEM((B * P, D), jnp.float32),
            pltpu.VMEM((D, B * P), jnp.float32),
        ],
    )(tokens, tokens_t, W, w_t, a3, a3t)
    patch_scores = acc.T.reshape(B, PH, PW)

    maps, scores = pl.pallas_call(
        _resize_kernel,
        grid=(B,),
        in_specs=[
            pl.BlockSpec((1, PH, PW), lambda b: (b, 0, 0)),
            pl.BlockSpec((H, PH), lambda b: (0, 0)),
        ],
        out_specs=[
            pl.BlockSpec((1, H, W_IMG), lambda b: (b, 0, 0)),
            pl.BlockSpec((B, 1), lambda b: (0, 0)),
        ],
        out_shape=[
            jax.ShapeDtypeStruct((B, H, W_IMG), jnp.float32),
            jax.ShapeDtypeStruct((B, 1), jnp.float32),
        ],
    )(patch_scores, rmat)
    return scores.reshape(B), maps


# speedup vs baseline: 31.1923x; 1.1207x over previous
import numpy as np
import jax
import jax.numpy as jnp
from jax.experimental import pallas as pl
from jax.experimental.pallas import tpu as pltpu

L = 2
B = 8
H = 224
W_IMG = 224
PATCH = 14
PH = 16
PW = 16
P = PH * PW
D = 1024
C_TOK = 3 * PATCH * PATCH


def _build_a3() -> np.ndarray:
    a = np.zeros((P, P), np.float32)
    for i in range(PH):
        for j in range(PW):
            p = i * PW + j
            nbrs = [(i + di, j + dj)
                    for di in (-1, 0, 1) for dj in (-1, 0, 1)
                    if 0 <= i + di < PH and 0 <= j + dj < PW]
            w = 1.0 / len(nbrs)
            for (y, x) in nbrs:
                a[p, y * PW + x] += w
    return a


def _build_resize_mat(n_in: int, n_out: int) -> np.ndarray:
    scale = n_in / n_out
    r = np.zeros((n_out, n_in), np.float32)
    for y in range(n_out):
        s = (y + 0.5) * scale - 0.5
        w = np.maximum(0.0, 1.0 - np.abs(s - np.arange(n_in)))
        r[y] = w / w.sum()
    return r.astype(np.float32)


_A3 = _build_a3()
_RMAT = _build_resize_mat(PH, H)


def _two_min_update(m1, m2, v):
    nm1 = jnp.minimum(m1, v)
    nm2 = jnp.minimum(m2, jnp.maximum(m1, v))
    return nm1, nm2


def _pair_msm(x_ref):
    big = jnp.float32(1e9)
    x2c = [jnp.sum(x_ref[b * P:(b + 1) * P, :] ** 2, axis=1, keepdims=True)
           for b in range(B)]
    x2r = [v.T for v in x2c]

    m1r = [jnp.full((P, 1), big) for _ in range(B)]
    m2r = [jnp.full((P, 1), big) for _ in range(B)]
    m1c = [jnp.full((1, P), big) for _ in range(B)]
    m2c = [jnp.full((1, P), big) for _ in range(B)]

    for b in range(B - 1):
        rows = x_ref[b * P:(b + 1) * P, :]
        others = x_ref[(b + 1) * P:, :]
        g = jax.lax.dot_general(rows, others, (((1,), (1,)), ((), ())),
                                preferred_element_type=jnp.float32)
        for j, c in enumerate(range(b + 1, B)):
            gc = g[:, j * P:(j + 1) * P]
            mb = jnp.min(x2r[c] - 2.0 * gc, axis=1, keepdims=True) + x2c[b]
            m1r[b], m2r[b] = _two_min_update(m1r[b], m2r[b], mb)
            mc = jnp.min(x2c[b] - 2.0 * gc, axis=0, keepdims=True) + x2r[c]
            m1c[c], m2c[c] = _two_min_update(m1c[c], m2c[c], mc)

    cols = []
    for b in range(B):
        m1ct = m1c[b].T
        m2ct = m2c[b].T
        m1 = jnp.minimum(m1r[b], m1ct)
        m2 = jnp.minimum(jnp.maximum(m1r[b], m1ct), jnp.minimum(m2r[b], m2ct))
        d1 = jnp.sqrt(jnp.maximum(m1, 1e-12))
        d2 = jnp.sqrt(jnp.maximum(m2, 1e-12))
        cols.append((d1 + d2) * 0.125)
    return jnp.concatenate(cols, axis=1)


def _fused_kernel(t2d_ref, w_ref, a3_ref, acc_ref, f_ref, f3_ref):
    l = pl.program_id(0)
    f_ref[...] = jnp.dot(t2d_ref[...], w_ref[0],
                         preferred_element_type=jnp.float32)
    for b in range(B):
        f3_ref[b * P:(b + 1) * P, :] = jnp.dot(
            a3_ref[...], f_ref[b * P:(b + 1) * P, :],
            preferred_element_type=jnp.float32)
    contrib = _pair_msm(f_ref) + _pair_msm(f3_ref)

    @pl.when(l == 0)
    def _():
        acc_ref[...] = contrib

    @pl.when(l != 0)
    def _():
        acc_ref[...] = acc_ref[...] + contrib


def _resize_kernel(s_ref, r_ref, maps_ref, score_ref):
    s = s_ref[0]
    rm = r_ref[...]
    tmp = jnp.dot(rm, s, preferred_element_type=jnp.float32)
    m = jax.lax.dot_general(tmp, rm, (((1,), (1,)), ((), ())),
                            preferred_element_type=jnp.float32)
    maps_ref[0] = m
    b = pl.program_id(0)
    onehot = (jax.lax.broadcasted_iota(jnp.int32, (B, 1), 0) == b
              ).astype(jnp.float32)
    contrib = jnp.max(m) * onehot

    @pl.when(b == 0)
    def _():
        score_ref[...] = contrib

    @pl.when(b != 0)
    def _():
        score_ref[...] = score_ref[...] + contrib


def kernel(pixel_values, W):
    tokens = pixel_values.reshape(B, 3, PH, PATCH, PW, PATCH)
    tokens = tokens.transpose(0, 2, 4, 1, 3, 5).reshape(B * P, C_TOK)

    a3 = jnp.asarray(_A3)
    rmat = jnp.asarray(_RMAT)

    acc = pl.pallas_call(
        _fused_kernel,
        grid=(L,),
        in_specs=[
            pl.BlockSpec((B * P, C_TOK), lambda l: (0, 0)),
            pl.BlockSpec((1, C_TOK, D), lambda l: (l, 0, 0)),
            pl.BlockSpec((P, P), lambda l: (0, 0)),
        ],
        out_specs=pl.BlockSpec((P, B), lambda l: (0, 0)),
        out_shape=jax.ShapeDtypeStruct((P, B), jnp.float32),
        scratch_shapes=[
            pltpu.VMEM((B * P, D), jnp.float32),
            pltpu.VMEM((B * P, D), jnp.float32),
        ],
    )(tokens, W, a3)
    patch_scores = acc.T.reshape(B, PH, PW)

    maps, scores = pl.pallas_call(
        _resize_kernel,
        grid=(B,),
        in_specs=[
            pl.BlockSpec((1, PH, PW), lambda b: (b, 0, 0)),
            pl.BlockSpec((H, PH), lambda b: (0, 0)),
        ],
        out_specs=[
            pl.BlockSpec((1, H, W_IMG), lambda b: (b, 0, 0)),
            pl.BlockSpec((B, 1), lambda b: (0, 0)),
        ],
        out_shape=[
            jax.ShapeDtypeStruct((B, H, W_IMG), jnp.float32),
            jax.ShapeDtypeStruct((B, 1), jnp.float32),
        ],
    )(patch_scores, rmat)
    return scores.reshape(B), maps
